# Initial kernel scaffold; baseline (speedup 1.0000x reference)
import functools

import jax
import jax.numpy as jnp
from jax import lax
from jax.experimental import pallas as pl
from jax.experimental.pallas import tpu as pltpu
from jax.experimental.pallas import tpu_sc as plsc

PLANE_RES = 1024
FEAT = 2
N_PTS = 1048576
SCALES = (128, 256, 512, 1024)
NW = 32
PPW = N_PTS // NW
C = 128
NG = C // 16
NCHUNK = PPW // C

_LEVEL_BASE = []
_acc = 0
for _s in SCALES:
    _LEVEL_BASE.append(_acc)
    _acc += 3 * _s * _s
R_TOTAL = _acc


def _build_quad_table(plane_embedding):
    t = plane_embedding.reshape(3, PLANE_RES, PLANE_RES, FEAT)
    qs = []
    for s in SCALES:
        m = PLANE_RES // s
        tb = jnp.concatenate([t[:, ::m], t[:, -1:]], axis=1)
        tt = jnp.concatenate([tb[:, :, ::m], tb[:, :, -1:]], axis=2)
        q = jnp.concatenate(
            [tt[:, :-1, :-1], tt[:, :-1, 1:], tt[:, 1:, :-1], tt[:, 1:, 1:]],
            axis=-1)
        qs.append(q.reshape(-1, 8))
    return jnp.concatenate(qs, axis=0)


def _sc_body(qtab, pos, out, pos_v, idx_v, gath_v, out_v, sem):
    wid = lax.axis_index("s") * 2 + lax.axis_index("c")
    lane = lax.iota(jnp.int32, 16)
    lane32 = lane * 32

    def chunk(ch, carry):
        base = wid * PPW + ch * C
        pltpu.sync_copy(pos.at[:, pl.ds(base, C)], pos_v)
        for g in range(NG):
            x = pos_v[0, pl.ds(g * 16, 16)]
            y = pos_v[1, pl.ds(g * 16, 16)]
            z = pos_v[2, pl.ds(g * 16, 16)]
            for l, s in enumerate(SCALES):
                pgs = []
                for v in (x, y, z):
                    p_ = v * (s - 1.0) + 0.5
                    pgs.append(jnp.floor(p_).astype(jnp.int32))
                for p, (a, b) in enumerate(((0, 1), (1, 2), (2, 0))):
                    row = pgs[b] * s + pgs[a] + (_LEVEL_BASE[l] + p * s * s)
                    idx_v[l * 3 + p, pl.ds(g * 16, 16)] = row
        copies = [pltpu.async_copy(qtab.at[idx_v.at[c]], gath_v.at[c], sem)
                  for c in range(12)]
        for cp in copies:
            cp.wait()
        for g in range(NG):
            x = pos_v[0, pl.ds(g * 16, 16)]
            y = pos_v[1, pl.ds(g * 16, 16)]
            z = pos_v[2, pl.ds(g * 16, 16)]
            pt = lane + g * 16
            for l, s in enumerate(SCALES):
                fr = []
                for v in (x, y, z):
                    p_ = v * (s - 1.0) + 0.5
                    fr.append(p_ - jnp.floor(p_))
                feats = []
                for p, (a, b) in enumerate(((0, 1), (1, 2), (2, 0))):
                    c = l * 3 + p
                    cvec = jnp.full((16,), c, jnp.int32)
                    k = [plsc.load_gather(
                            gath_v, [cvec, pt, jnp.full((16,), kk, jnp.int32)])
                         for kk in range(8)]
                    fa, fb = fr[a], fr[b]
                    m00 = k[0] + fa * (k[2] - k[0])
                    m01 = k[1] + fa * (k[3] - k[1])
                    m10 = k[4] + fa * (k[6] - k[4])
                    m11 = k[5] + fa * (k[7] - k[5])
                    feats.append((m00 + fb * (m10 - m00),
                                  m01 + fb * (m11 - m01)))
                prod0 = feats[0][0] * feats[1][0] * feats[2][0]
                prod1 = feats[0][1] * feats[1][1] * feats[2][1]
                cols = [feats[0][0], feats[0][1], feats[1][0], feats[1][1],
                        feats[2][0], feats[2][1], prod0, prod1]
                for ci, vec in enumerate(cols):
                    plsc.store_scatter(
                        out_v, [lane32 + (g * 512 + l * 8 + ci)], vec)
        pltpu.sync_copy(out_v, out.at[pl.ds(base * 32, C * 32)])
        return carry

    lax.fori_loop(0, NCHUNK, chunk, 0)


@functools.partial(jax.jit)
def kernel(positions, plane_embedding):
    qtab = _build_quad_table(plane_embedding)
    pos_t = positions.T
    mesh = plsc.VectorSubcoreMesh(core_axis_name="c", subcore_axis_name="s")
    f = pl.kernel(
        _sc_body,
        out_type=jax.ShapeDtypeStruct((N_PTS * 32,), jnp.float32),
        mesh=mesh,
        scratch_types=[
            pltpu.VMEM((3, C), jnp.float32),
            pltpu.VMEM((12, C), jnp.int32),
            pltpu.VMEM((12, C, 8), jnp.float32),
            pltpu.VMEM((C * 32,), jnp.float32),
            pltpu.SemaphoreType.DMA,
        ],
    )
    return f(qtab, pos_t).reshape(N_PTS, 32)


# speedup vs baseline: 15.8179x; 15.8179x over previous
import functools

import jax
import jax.numpy as jnp
from jax import lax
from jax.experimental import pallas as pl
from jax.experimental.pallas import tpu as pltpu
from jax.experimental.pallas import tpu_sc as plsc

PLANE_RES = 1024
FEAT = 2
N_PTS = 1048576
SCALES = (128, 256, 512, 1024)
NW = 32
PPW = N_PTS // NW
C = 128
NG = C // 16
NCHUNK = PPW // C

_LEVEL_BASE = []
_acc = 0
for _s in SCALES:
    _LEVEL_BASE.append(_acc)
    _acc += 3 * _s * _s
R_TOTAL = _acc


def _build_quad_table(plane_embedding):
    t = plane_embedding.reshape(3, PLANE_RES, PLANE_RES, FEAT)
    qs = []
    for s in SCALES:
        m = PLANE_RES // s
        tb = jnp.concatenate([t[:, ::m], t[:, -1:]], axis=1)
        tt = jnp.concatenate([tb[:, :, ::m], tb[:, :, -1:]], axis=2)
        q = jnp.concatenate(
            [tt[:, :-1, :-1], tt[:, :-1, 1:], tt[:, 1:, :-1], tt[:, 1:, 1:]],
            axis=-1)
        qs.append(q.reshape(-1, 8))
    return jnp.concatenate(qs, axis=0)


def _sc_body(qtab, pos, out, pos_v, idx_v, gath_v, out_v, sem):
    wid = lax.axis_index("s") * 2 + lax.axis_index("c")
    lane = lax.iota(jnp.int32, 16)
    lane32 = lane * 32

    def chunk(ch, carry):
        base = wid * PPW + ch * C
        pltpu.sync_copy(pos.at[:, pl.ds(base, C)], pos_v)
        for g in range(NG):
            x = pos_v[0, pl.ds(g * 16, 16)]
            y = pos_v[1, pl.ds(g * 16, 16)]
            z = pos_v[2, pl.ds(g * 16, 16)]
            for l, s in enumerate(SCALES):
                pgs = []
                for v in (x, y, z):
                    p_ = v * (s - 1.0) + 0.5
                    pgs.append(p_.astype(jnp.int32))
                for p, (a, b) in enumerate(((0, 1), (1, 2), (2, 0))):
                    row = pgs[b] * s + pgs[a] + (_LEVEL_BASE[l] + p * s * s)
                    idx_v[l * 3 + p, pl.ds(g * 16, 16)] = row
        copies = [pltpu.async_copy(qtab.at[idx_v.at[c]],
                                   gath_v.at[pl.ds(c * C, C)], sem)
                  for c in range(12)]
        for cp in copies:
            cp.wait()
        for g in range(NG):
            x = pos_v[0, pl.ds(g * 16, 16)]
            y = pos_v[1, pl.ds(g * 16, 16)]
            z = pos_v[2, pl.ds(g * 16, 16)]
            pt = lane + g * 16
            for l, s in enumerate(SCALES):
                fr = []
                for v in (x, y, z):
                    p_ = v * (s - 1.0) + 0.5
                    fr.append(p_ - p_.astype(jnp.int32).astype(jnp.float32))
                feats = []
                for p, (a, b) in enumerate(((0, 1), (1, 2), (2, 0))):
                    c = l * 3 + p
                    rvec = pt + c * C
                    k = [plsc.load_gather(
                            gath_v, [rvec, jnp.full((16,), kk, jnp.int32)])
                         for kk in range(8)]
                    fa, fb = fr[a], fr[b]
                    m00 = k[0] + fa * (k[2] - k[0])
                    m01 = k[1] + fa * (k[3] - k[1])
                    m10 = k[4] + fa * (k[6] - k[4])
                    m11 = k[5] + fa * (k[7] - k[5])
                    feats.append((m00 + fb * (m10 - m00),
                                  m01 + fb * (m11 - m01)))
                prod0 = feats[0][0] * feats[1][0] * feats[2][0]
                prod1 = feats[0][1] * feats[1][1] * feats[2][1]
                cols = [feats[0][0], feats[0][1], feats[1][0], feats[1][1],
                        feats[2][0], feats[2][1], prod0, prod1]
                for ci, vec in enumerate(cols):
                    plsc.store_scatter(
                        out_v, [lane32 + (g * 512 + l * 8 + ci)], vec)
        pltpu.sync_copy(out_v, out.at[pl.ds(base * 32, C * 32)])
        return carry

    lax.fori_loop(0, NCHUNK, chunk, 0)


@functools.partial(jax.jit)
def kernel(positions, plane_embedding):
    qtab = _build_quad_table(plane_embedding)
    pos_t = positions.T
    mesh = plsc.VectorSubcoreMesh(core_axis_name="c", subcore_axis_name="s")
    f = pl.kernel(
        _sc_body,
        out_type=jax.ShapeDtypeStruct((N_PTS * 32,), jnp.float32),
        mesh=mesh,
        compiler_params=pltpu.CompilerParams(
            needs_layout_passes=False, use_tc_tiling_on_sc=False),
        scratch_types=[
            pltpu.VMEM((3, C), jnp.float32),
            pltpu.VMEM((12, C), jnp.int32),
            pltpu.VMEM((12 * C, 8), jnp.float32),
            pltpu.VMEM((C * 32,), jnp.float32),
            pltpu.SemaphoreType.DMA,
        ],
    )
    return f(qtab, pos_t).reshape(N_PTS, 32)


# speedup vs baseline: 35.9210x; 2.2709x over previous
import functools

import jax
import jax.numpy as jnp
from jax import lax
from jax.experimental import pallas as pl
from jax.experimental.pallas import tpu as pltpu
from jax.experimental.pallas import tpu_sc as plsc

PLANE_RES = 1024
FEAT = 2
N_PTS = 1048576
SCALES = (128, 256, 512, 1024)
NW = 32
PPW = N_PTS // NW
C = 128
NG = C // 16
NCHUNK = PPW // C

_LEVEL_BASE = []
_acc = 0
for _s in SCALES:
    _LEVEL_BASE.append(_acc)
    _acc += 3 * _s * _s
R_TOTAL = _acc

_PLANE_SZ = PLANE_RES * PLANE_RES * FEAT


def _build_body(pe, qt, rows_v, out_v, sem):
    wid = lax.axis_index("s") * 2 + lax.axis_index("c")
    lane = lax.iota(jnp.int32, 16)
    qdel = lane // 8
    adel = (lane // 2) % 2
    fplus = (lane % 2) + (lane // 8) * 0
    rowsel = (lane % 8) // 4 * 2048

    for l, s in enumerate(SCALES):
        m = PLANE_RES // s
        n_l = 3 * s // NW

        def stripe(j, carry, l=l, s=s, m=m, n_l=n_l):
            sid = wid * n_l + j
            p = sid // s
            gb = sid % s
            rowa = jnp.minimum(gb * m, 1023)
            rowb = jnp.minimum((gb + 1) * m, 1023)
            pltpu.sync_copy(
                pe.at[pl.ds(p * _PLANE_SZ + rowa * 2048, 2048)],
                rows_v.at[pl.ds(0, 2048)])
            pltpu.sync_copy(
                pe.at[pl.ds(p * _PLANE_SZ + rowb * 2048, 2048)],
                rows_v.at[pl.ds(2048, 2048)])

            kcol = lane % 8

            def pair(i, carry2):
                q = i * 2 + qdel
                la = jnp.minimum((q + adel) * m, 1023)
                idx = la * 2 + fplus + rowsel
                plsc.store_scatter(out_v, [i * 2 + qdel, kcol],
                                   plsc.load_gather(rows_v, [idx]))
                return carry2

            lax.fori_loop(0, s // 2, pair, 0)
            row0 = _LEVEL_BASE[l] + (p * s + gb) * s
            pltpu.sync_copy(out_v.at[pl.ds(0, s), :],
                            qt.at[pl.ds(row0, s), :])
            return carry

        lax.fori_loop(0, n_l, stripe, 0)


def _lookup_body(qtab, pos, out, pos_v, idx_v, gath_v, out_v, sem):
    wid = lax.axis_index("s") * 2 + lax.axis_index("c")
    lane = lax.iota(jnp.int32, 16)
    lane3 = lane * 3
    lane32 = lane * 32

    def chunk(ch, carry):
        base = wid * PPW + ch * C
        pltpu.sync_copy(pos.at[pl.ds(base * 3, C * 3)], pos_v)
        for g in range(NG):
            x = plsc.load_gather(pos_v, [lane3 + g * 48])
            y = plsc.load_gather(pos_v, [lane3 + (g * 48 + 1)])
            z = plsc.load_gather(pos_v, [lane3 + (g * 48 + 2)])
            for l, s in enumerate(SCALES):
                pgs = []
                for v in (x, y, z):
                    p_ = v * (s - 1.0) + 0.5
                    pgs.append(p_.astype(jnp.int32))
                for p, (a, b) in enumerate(((0, 1), (1, 2), (2, 0))):
                    row = pgs[b] * s + pgs[a] + (_LEVEL_BASE[l] + p * s * s)
                    idx_v[l * 3 + p, pl.ds(g * 16, 16)] = row
        copies = [pltpu.async_copy(qtab.at[idx_v.at[c]],
                                   gath_v.at[pl.ds(c * C, C)], sem)
                  for c in range(12)]
        for cp in copies:
            cp.wait()
        for g in range(NG):
            x = plsc.load_gather(pos_v, [lane3 + g * 48])
            y = plsc.load_gather(pos_v, [lane3 + (g * 48 + 1)])
            z = plsc.load_gather(pos_v, [lane3 + (g * 48 + 2)])
            pt = lane + g * 16
            for l, s in enumerate(SCALES):
                fr = []
                for v in (x, y, z):
                    p_ = v * (s - 1.0) + 0.5
                    fr.append(p_ - p_.astype(jnp.int32).astype(jnp.float32))
                feats = []
                for p, (a, b) in enumerate(((0, 1), (1, 2), (2, 0))):
                    c = l * 3 + p
                    rvec = pt + c * C
                    k = [plsc.load_gather(
                            gath_v, [rvec, jnp.full((16,), kk, jnp.int32)])
                         for kk in range(8)]
                    fa, fb = fr[a], fr[b]
                    m00 = k[0] + fa * (k[2] - k[0])
                    m01 = k[1] + fa * (k[3] - k[1])
                    m10 = k[4] + fa * (k[6] - k[4])
                    m11 = k[5] + fa * (k[7] - k[5])
                    feats.append((m00 + fb * (m10 - m00),
                                  m01 + fb * (m11 - m01)))
                prod0 = feats[0][0] * feats[1][0] * feats[2][0]
                prod1 = feats[0][1] * feats[1][1] * feats[2][1]
                cols = [feats[0][0], feats[0][1], feats[1][0], feats[1][1],
                        feats[2][0], feats[2][1], prod0, prod1]
                for ci, vec in enumerate(cols):
                    plsc.store_scatter(
                        out_v, [lane32 + (g * 512 + l * 8 + ci)], vec)
        pltpu.sync_copy(out_v, out.at[pl.ds(base * 32, C * 32)])
        return carry

    lax.fori_loop(0, NCHUNK, chunk, 0)


@functools.partial(jax.jit)
def kernel(positions, plane_embedding):
    mesh = plsc.VectorSubcoreMesh(core_axis_name="c", subcore_axis_name="s")
    cparams = pltpu.CompilerParams(
        needs_layout_passes=False, use_tc_tiling_on_sc=False)
    build = pl.kernel(
        _build_body,
        out_type=jax.ShapeDtypeStruct((R_TOTAL, 8), jnp.float32),
        mesh=mesh,
        compiler_params=cparams,
        scratch_types=[
            pltpu.VMEM((4096,), jnp.float32),
            pltpu.VMEM((1024, 8), jnp.float32),
            pltpu.SemaphoreType.DMA,
        ],
    )
    lookup = pl.kernel(
        _lookup_body,
        out_type=jax.ShapeDtypeStruct((N_PTS * 32,), jnp.float32),
        mesh=mesh,
        compiler_params=cparams,
        scratch_types=[
            pltpu.VMEM((3 * C,), jnp.float32),
            pltpu.VMEM((12, C), jnp.int32),
            pltpu.VMEM((12 * C, 8), jnp.float32),
            pltpu.VMEM((C * 32,), jnp.float32),
            pltpu.SemaphoreType.DMA,
        ],
    )
    qtab = build(plane_embedding)
    return lookup(qtab, positions.reshape(-1)).reshape(N_PTS, 32)


# speedup vs baseline: 42.9559x; 1.1958x over previous
import functools

import jax
import jax.numpy as jnp
from jax import lax
from jax.experimental import pallas as pl
from jax.experimental.pallas import tpu as pltpu
from jax.experimental.pallas import tpu_sc as plsc

PLANE_RES = 1024
FEAT = 2
N_PTS = 1048576
SCALES = (128, 256, 512, 1024)
NW = 32
PPW = N_PTS // NW
C = 128
NG = C // 16
NCHUNK = PPW // C

_LEVEL_BASE = []
_acc = 0
for _s in SCALES:
    _LEVEL_BASE.append(_acc)
    _acc += 3 * _s * _s
R_TOTAL = _acc

_PLANE_SZ = PLANE_RES * PLANE_RES * FEAT


def _build_body(pe, qt, rows_v, out_v, sem):
    wid = lax.axis_index("s") * 2 + lax.axis_index("c")
    lane = lax.iota(jnp.int32, 16)
    qdel = lane // 8
    adel = (lane // 2) % 2
    fplus = (lane % 2) + (lane // 8) * 0
    rowsel = (lane % 8) // 4 * 2048

    for l, s in enumerate(SCALES):
        m = PLANE_RES // s
        n_l = 3 * s // NW

        def stripe(j, carry, l=l, s=s, m=m, n_l=n_l):
            sid = wid * n_l + j
            p = sid // s
            gb = sid % s
            rowa = jnp.minimum(gb * m, 1023)
            rowb = jnp.minimum((gb + 1) * m, 1023)
            pltpu.sync_copy(
                pe.at[pl.ds(p * _PLANE_SZ + rowa * 2048, 2048)],
                rows_v.at[pl.ds(0, 2048)])
            pltpu.sync_copy(
                pe.at[pl.ds(p * _PLANE_SZ + rowb * 2048, 2048)],
                rows_v.at[pl.ds(2048, 2048)])

            kcol = lane % 8

            def pair(i, carry2):
                q = i * 2 + qdel
                la = jnp.minimum((q + adel) * m, 1023)
                idx = la * 2 + fplus + rowsel
                plsc.store_scatter(out_v, [i * 2 + qdel, kcol],
                                   plsc.load_gather(rows_v, [idx]))
                return carry2

            lax.fori_loop(0, s // 2, pair, 0)
            row0 = _LEVEL_BASE[l] + (p * s + gb) * s
            pltpu.sync_copy(out_v.at[pl.ds(0, s), :],
                            qt.at[pl.ds(row0, s), :])
            return carry

        lax.fori_loop(0, n_l, stripe, 0)


DCH = 256
DNC = PPW // DCH


def _depad_body(posn, pflat, buf0, buf1, out0, out1, sem_i, sem_o):
    wid = lax.axis_index("s") * 2 + lax.axis_index("c")
    lane = lax.iota(jnp.int32, 16)
    base0 = wid * PPW
    bufs = (buf0, buf1)
    outs = (out0, out1)

    def in_copy(ch, slot):
        return pltpu.make_async_copy(
            posn.at[pl.ds(base0 + ch * DCH, DCH), :], bufs[slot], sem_i)

    def out_copy(ch, slot):
        return pltpu.make_async_copy(
            outs[slot],
            pflat.at[pl.ds((base0 + ch * DCH) * 3, DCH * 3)], sem_o)

    in_copy(0, 0).start()
    in_copy(1, 1).start()

    def pair(j, carry):
        for b in range(2):
            ch = j * 2 + b
            in_copy(ch, b).wait()

            @pl.when(ch >= 2)
            def _():
                out_copy(ch - 2, b).wait()

            for g in range(DCH // 16):
                pv = lane + g * 16
                for c in range(3):
                    v = plsc.load_gather(
                        bufs[b], [pv, jnp.full((16,), c, jnp.int32)])
                    plsc.store_scatter(
                        outs[b], [lane * 3 + (g * 48 + c)], v)
            out_copy(ch, b).start()

            @pl.when(ch + 2 < DNC)
            def _():
                in_copy(ch + 2, b).start()
        return carry

    lax.fori_loop(0, DNC // 2, pair, 0)
    out_copy(DNC - 2, 0).wait()
    out_copy(DNC - 1, 1).wait()


def _lookup_body(qtab, pos, out, pos_v, idx_v, gath_v, out_v, sem):
    wid = lax.axis_index("s") * 2 + lax.axis_index("c")
    lane = lax.iota(jnp.int32, 16)
    lane3 = lane * 3
    lane32 = lane * 32

    def chunk(ch, carry):
        base = wid * PPW + ch * C
        pltpu.sync_copy(pos.at[pl.ds(base * 3, C * 3)], pos_v)
        for g in range(NG):
            x = plsc.load_gather(pos_v, [lane3 + g * 48])
            y = plsc.load_gather(pos_v, [lane3 + (g * 48 + 1)])
            z = plsc.load_gather(pos_v, [lane3 + (g * 48 + 2)])
            for l, s in enumerate(SCALES):
                pgs = []
                for v in (x, y, z):
                    p_ = v * (s - 1.0) + 0.5
                    pgs.append(p_.astype(jnp.int32))
                for p, (a, b) in enumerate(((0, 1), (1, 2), (2, 0))):
                    row = pgs[b] * s + pgs[a] + (_LEVEL_BASE[l] + p * s * s)
                    idx_v[l * 3 + p, pl.ds(g * 16, 16)] = row
        copies = [pltpu.async_copy(qtab.at[idx_v.at[c]],
                                   gath_v.at[pl.ds(c * C, C)], sem)
                  for c in range(12)]
        for cp in copies:
            cp.wait()
        for g in range(NG):
            x = plsc.load_gather(pos_v, [lane3 + g * 48])
            y = plsc.load_gather(pos_v, [lane3 + (g * 48 + 1)])
            z = plsc.load_gather(pos_v, [lane3 + (g * 48 + 2)])
            pt = lane + g * 16
            for l, s in enumerate(SCALES):
                fr = []
                for v in (x, y, z):
                    p_ = v * (s - 1.0) + 0.5
                    fr.append(p_ - p_.astype(jnp.int32).astype(jnp.float32))
                feats = []
                for p, (a, b) in enumerate(((0, 1), (1, 2), (2, 0))):
                    c = l * 3 + p
                    rvec = pt + c * C
                    k = [plsc.load_gather(
                            gath_v, [rvec, jnp.full((16,), kk, jnp.int32)])
                         for kk in range(8)]
                    fa, fb = fr[a], fr[b]
                    m00 = k[0] + fa * (k[2] - k[0])
                    m01 = k[1] + fa * (k[3] - k[1])
                    m10 = k[4] + fa * (k[6] - k[4])
                    m11 = k[5] + fa * (k[7] - k[5])
                    feats.append((m00 + fb * (m10 - m00),
                                  m01 + fb * (m11 - m01)))
                prod0 = feats[0][0] * feats[1][0] * feats[2][0]
                prod1 = feats[0][1] * feats[1][1] * feats[2][1]
                cols = [feats[0][0], feats[0][1], feats[1][0], feats[1][1],
                        feats[2][0], feats[2][1], prod0, prod1]
                for ci, vec in enumerate(cols):
                    plsc.store_scatter(
                        out_v, [lane32 + (g * 512 + l * 8 + ci)], vec)
        pltpu.sync_copy(out_v, out.at[pl.ds(base * 32, C * 32)])
        return carry

    lax.fori_loop(0, NCHUNK, chunk, 0)


@functools.partial(jax.jit)
def kernel(positions, plane_embedding):
    mesh = plsc.VectorSubcoreMesh(core_axis_name="c", subcore_axis_name="s")
    cparams = pltpu.CompilerParams(
        needs_layout_passes=False, use_tc_tiling_on_sc=False)
    build = pl.kernel(
        _build_body,
        out_type=jax.ShapeDtypeStruct((R_TOTAL, 8), jnp.float32),
        mesh=mesh,
        compiler_params=cparams,
        scratch_types=[
            pltpu.VMEM((4096,), jnp.float32),
            pltpu.VMEM((1024, 8), jnp.float32),
            pltpu.SemaphoreType.DMA,
        ],
    )
    lookup = pl.kernel(
        _lookup_body,
        out_type=jax.ShapeDtypeStruct((N_PTS * 32,), jnp.float32),
        mesh=mesh,
        compiler_params=cparams,
        scratch_types=[
            pltpu.VMEM((3 * C,), jnp.float32),
            pltpu.VMEM((12, C), jnp.int32),
            pltpu.VMEM((12 * C, 8), jnp.float32),
            pltpu.VMEM((C * 32,), jnp.float32),
            pltpu.SemaphoreType.DMA,
        ],
    )
    depad = pl.kernel(
        _depad_body,
        out_type=jax.ShapeDtypeStruct((N_PTS * 3,), jnp.float32),
        mesh=mesh,
        compiler_params=pltpu.CompilerParams(
            needs_layout_passes=False, use_tc_tiling_on_sc=True),
        scratch_types=[
            pltpu.VMEM((DCH, 3), jnp.float32),
            pltpu.VMEM((DCH, 3), jnp.float32),
            pltpu.VMEM((DCH * 3,), jnp.float32),
            pltpu.VMEM((DCH * 3,), jnp.float32),
            pltpu.SemaphoreType.DMA,
            pltpu.SemaphoreType.DMA,
        ],
    )
    qtab = build(plane_embedding)
    return lookup(qtab, depad(positions)).reshape(N_PTS, 32)


# speedup vs baseline: 54.6551x; 1.2724x over previous
import functools

import jax
import jax.numpy as jnp
from jax import lax
from jax.experimental import pallas as pl
from jax.experimental.pallas import tpu as pltpu
from jax.experimental.pallas import tpu_sc as plsc

PLANE_RES = 1024
FEAT = 2
N_PTS = 1048576
SCALES = (128, 256, 512, 1024)
NW = 32
PPW = N_PTS // NW
C = 128
NG = C // 16
NCHUNK = PPW // C

_LEVEL_BASE = []
_acc = 0
for _s in SCALES:
    _LEVEL_BASE.append(_acc)
    _acc += 3 * _s * _s
R_TOTAL = _acc

_PLANE_SZ = PLANE_RES * PLANE_RES * FEAT


def _build_body(pe, qt, rows_v, out_v, sem):
    wid = lax.axis_index("s") * 2 + lax.axis_index("c")
    lane = lax.iota(jnp.int32, 16)
    qdel = lane // 8
    adel = (lane // 2) % 2
    fplus = (lane % 2) + (lane // 8) * 0
    rowsel = (lane % 8) // 4 * 2048

    for l, s in enumerate(SCALES):
        m = PLANE_RES // s
        n_l = 3 * s // NW

        def stripe(j, carry, l=l, s=s, m=m, n_l=n_l):
            sid = wid * n_l + j
            p = sid // s
            gb = sid % s
            rowa = jnp.minimum(gb * m, 1023)
            rowb = jnp.minimum((gb + 1) * m, 1023)
            pltpu.sync_copy(
                pe.at[pl.ds(p * _PLANE_SZ + rowa * 2048, 2048)],
                rows_v.at[pl.ds(0, 2048)])
            pltpu.sync_copy(
                pe.at[pl.ds(p * _PLANE_SZ + rowb * 2048, 2048)],
                rows_v.at[pl.ds(2048, 2048)])

            kcol = lane % 8

            def pair(i, carry2):
                q = i * 2 + qdel
                la = jnp.minimum((q + adel) * m, 1023)
                idx = la * 2 + fplus + rowsel
                plsc.store_scatter(out_v, [i * 2 + qdel, kcol],
                                   plsc.load_gather(rows_v, [idx]))
                return carry2

            lax.fori_loop(0, s // 2, pair, 0)
            row0 = _LEVEL_BASE[l] + (p * s + gb) * s
            pltpu.sync_copy(out_v.at[pl.ds(0, s), :],
                            qt.at[pl.ds(row0, s), :])
            return carry

        lax.fori_loop(0, n_l, stripe, 0)


DCH = 256
DNC = PPW // DCH


def _depad_body(posn, pflat, buf0, buf1, out0, out1, sem_i, sem_o):
    wid = lax.axis_index("s") * 2 + lax.axis_index("c")
    lane = lax.iota(jnp.int32, 16)
    base0 = wid * PPW
    bufs = (buf0, buf1)
    outs = (out0, out1)

    def in_copy(ch, slot):
        return pltpu.make_async_copy(
            posn.at[pl.ds(base0 + ch * DCH, DCH), :], bufs[slot], sem_i)

    def out_copy(ch, slot):
        return pltpu.make_async_copy(
            outs[slot],
            pflat.at[pl.ds((base0 + ch * DCH) * 3, DCH * 3)], sem_o)

    in_copy(0, 0).start()
    in_copy(1, 1).start()

    def pair(j, carry):
        for b in range(2):
            ch = j * 2 + b
            in_copy(ch, b).wait()

            @pl.when(ch >= 2)
            def _():
                out_copy(ch - 2, b).wait()

            for g in range(DCH // 16):
                pv = lane + g * 16
                for c in range(3):
                    v = plsc.load_gather(
                        bufs[b], [pv, jnp.full((16,), c, jnp.int32)])
                    plsc.store_scatter(
                        outs[b], [lane * 3 + (g * 48 + c)], v)
            out_copy(ch, b).start()

            @pl.when(ch + 2 < DNC)
            def _():
                in_copy(ch + 2, b).start()
        return carry

    lax.fori_loop(0, DNC // 2, pair, 0)
    out_copy(DNC - 2, 0).wait()
    out_copy(DNC - 1, 1).wait()


def _lookup_body(qtab, pos, out, pv0, pv1, ix0, ix1, gv0, gv1, ov0, ov1,
                 sp0, sp1, sg0, sg1, so0, so1):
    wid = lax.axis_index("s") * 2 + lax.axis_index("c")
    lane = lax.iota(jnp.int32, 16)
    lane3 = lane * 3
    lane32 = lane * 32
    base0 = wid * PPW
    pvs, ixs, gvs, ovs = (pv0, pv1), (ix0, ix1), (gv0, gv1), (ov0, ov1)
    sps, sgs, sos = (sp0, sp1), (sg0, sg1), (so0, so1)

    def pos_cp(ch, b):
        return pltpu.make_async_copy(
            pos.at[pl.ds((base0 + ch * C) * 3, C * 3)], pvs[b], sps[b])

    def out_cp(ch, b):
        return pltpu.make_async_copy(
            ovs[b], out.at[pl.ds((base0 + ch * C) * 32, C * 32)], sos[b])

    def gath_cps(b):
        return [pltpu.make_async_copy(
                    qtab.at[ixs[b].at[c]], gvs[b].at[pl.ds(c * C, C)], sgs[b])
                for c in range(12)]

    def phase_a(b):
        pv, ix = pvs[b], ixs[b]

        def ga_body(g, carry):
            x = plsc.load_gather(pv, [lane3 + g * 48])
            y = plsc.load_gather(pv, [lane3 + (g * 48 + 1)])
            z = plsc.load_gather(pv, [lane3 + (g * 48 + 2)])
            for l, s in enumerate(SCALES):
                pgs = []
                for v in (x, y, z):
                    p_ = v * (s - 1.0) + 0.5
                    pgs.append(p_.astype(jnp.int32))
                for p, (a, b2) in enumerate(((0, 1), (1, 2), (2, 0))):
                    row = pgs[b2] * s + pgs[a] + (_LEVEL_BASE[l] + p * s * s)
                    ix[l * 3 + p, pl.ds(g * 16, 16)] = row
            return carry

        lax.fori_loop(0, NG, ga_body, 0)

    def phase_c(b):
        pv, gv, ov = pvs[b], gvs[b], ovs[b]

        def gc_body(g, carry):
            x = plsc.load_gather(pv, [lane3 + g * 48])
            y = plsc.load_gather(pv, [lane3 + (g * 48 + 1)])
            z = plsc.load_gather(pv, [lane3 + (g * 48 + 2)])
            pt = lane + g * 16
            for l, s in enumerate(SCALES):
                fr = []
                for v in (x, y, z):
                    p_ = v * (s - 1.0) + 0.5
                    fr.append(p_ - p_.astype(jnp.int32).astype(jnp.float32))
                feats = []
                for p, (a, b2) in enumerate(((0, 1), (1, 2), (2, 0))):
                    c = l * 3 + p
                    rvec = pt + c * C
                    k = [plsc.load_gather(
                            gv, [rvec, jnp.full((16,), kk, jnp.int32)])
                         for kk in range(8)]
                    fa, fb = fr[a], fr[b2]
                    m00 = k[0] + fa * (k[2] - k[0])
                    m01 = k[1] + fa * (k[3] - k[1])
                    m10 = k[4] + fa * (k[6] - k[4])
                    m11 = k[5] + fa * (k[7] - k[5])
                    feats.append((m00 + fb * (m10 - m00),
                                  m01 + fb * (m11 - m01)))
                prod0 = feats[0][0] * feats[1][0] * feats[2][0]
                prod1 = feats[0][1] * feats[1][1] * feats[2][1]
                cols = [feats[0][0], feats[0][1], feats[1][0], feats[1][1],
                        feats[2][0], feats[2][1], prod0, prod1]
                for ci, vec in enumerate(cols):
                    plsc.store_scatter(
                        ov, [lane32 + (g * 512 + l * 8 + ci)], vec)
            return carry

        lax.fori_loop(0, NG, gc_body, 0)

    pos_cp(0, 0).start()
    pos_cp(0, 0).wait()
    phase_a(0)
    for cp in gath_cps(0):
        cp.start()
    pos_cp(1, 1).start()

    def pair(j, carry):
        for b2 in range(2):
            ch = 1 + j * 2 + b2
            s = (1 + b2) % 2
            o = 1 - s
            pos_cp(ch, s).wait()
            phase_a(s)
            for cp in gath_cps(s):
                cp.start()
            for cp in gath_cps(o):
                cp.wait()

            @pl.when(ch >= 3)
            def _():
                out_cp(ch - 3, o).wait()

            phase_c(o)
            out_cp(ch - 1, o).start()
            pos_cp(ch + 1, o).start()
        return carry

    lax.fori_loop(0, (NCHUNK - 2) // 2, pair, 0)
    last = NCHUNK - 1
    pos_cp(last, 1).wait()
    phase_a(1)
    for cp in gath_cps(1):
        cp.start()
    for cp in gath_cps(0):
        cp.wait()
    out_cp(last - 3, 0).wait()
    phase_c(0)
    out_cp(last - 1, 0).start()
    for cp in gath_cps(1):
        cp.wait()
    out_cp(last - 2, 1).wait()
    phase_c(1)
    out_cp(last, 1).start()
    out_cp(last - 1, 0).wait()
    out_cp(last, 1).wait()


@functools.partial(jax.jit)
def kernel(positions, plane_embedding):
    mesh = plsc.VectorSubcoreMesh(core_axis_name="c", subcore_axis_name="s")
    cparams = pltpu.CompilerParams(
        needs_layout_passes=False, use_tc_tiling_on_sc=False)
    build = pl.kernel(
        _build_body,
        out_type=jax.ShapeDtypeStruct((R_TOTAL, 8), jnp.float32),
        mesh=mesh,
        compiler_params=cparams,
        scratch_types=[
            pltpu.VMEM((4096,), jnp.float32),
            pltpu.VMEM((1024, 8), jnp.float32),
            pltpu.SemaphoreType.DMA,
        ],
    )
    lookup = pl.kernel(
        _lookup_body,
        out_type=jax.ShapeDtypeStruct((N_PTS * 32,), jnp.float32),
        mesh=mesh,
        compiler_params=cparams,
        scratch_types=[
            pltpu.VMEM((3 * C,), jnp.float32),
            pltpu.VMEM((3 * C,), jnp.float32),
            pltpu.VMEM((12, C), jnp.int32),
            pltpu.VMEM((12, C), jnp.int32),
            pltpu.VMEM((12 * C, 8), jnp.float32),
            pltpu.VMEM((12 * C, 8), jnp.float32),
            pltpu.VMEM((C * 32,), jnp.float32),
            pltpu.VMEM((C * 32,), jnp.float32),
            pltpu.SemaphoreType.DMA,
            pltpu.SemaphoreType.DMA,
            pltpu.SemaphoreType.DMA,
            pltpu.SemaphoreType.DMA,
            pltpu.SemaphoreType.DMA,
            pltpu.SemaphoreType.DMA,
        ],
    )
    depad = pl.kernel(
        _depad_body,
        out_type=jax.ShapeDtypeStruct((N_PTS * 3,), jnp.float32),
        mesh=mesh,
        compiler_params=pltpu.CompilerParams(
            needs_layout_passes=False, use_tc_tiling_on_sc=True),
        scratch_types=[
            pltpu.VMEM((DCH, 3), jnp.float32),
            pltpu.VMEM((DCH, 3), jnp.float32),
            pltpu.VMEM((DCH * 3,), jnp.float32),
            pltpu.VMEM((DCH * 3,), jnp.float32),
            pltpu.SemaphoreType.DMA,
            pltpu.SemaphoreType.DMA,
        ],
    )
    qtab = build(plane_embedding)
    return lookup(qtab, depad(positions)).reshape(N_PTS, 32)


# speedup vs baseline: 61.7850x; 1.1305x over previous
import functools

import jax
import jax.numpy as jnp
from jax import lax
from jax.experimental import pallas as pl
from jax.experimental.pallas import tpu as pltpu
from jax.experimental.pallas import tpu_sc as plsc

PLANE_RES = 1024
FEAT = 2
N_PTS = 1048576
SCALES = (128, 256, 512, 1024)
NW = 32
PPW = N_PTS // NW
C = 128
NG = C // 16
NCHUNK = PPW // C

_LEVEL_BASE = []
_acc = 0
for _s in SCALES:
    _LEVEL_BASE.append(_acc)
    _acc += 3 * _s * _s
R_TOTAL = _acc

_PLANE_SZ = PLANE_RES * PLANE_RES * FEAT


def _build_body(pe, qt, ra0, ra1, ob0, ob1, sem_i0, sem_i1, sem_o0, sem_o1):
    wid = lax.axis_index("s") * 2 + lax.axis_index("c")
    lane = lax.iota(jnp.int32, 16)
    qdel = lane // 8
    adel = (lane // 2) % 2
    fplus = (lane % 2) + (lane // 8) * 0
    rowsel = (lane % 8) // 4 * 2048

    kcol = lane % 8
    rbufs = (ra0, ra1)
    obufs = (ob0, ob1)
    sis = (sem_i0, sem_i1)
    sos = (sem_o0, sem_o1)

    for l, s in enumerate(SCALES):
        m = PLANE_RES // s
        n_l = 3 * s // NW

        def pg(j, s=s, m=m):
            sid = wid * n_l + j
            p = sid // s
            gb = sid % s
            return p, gb

        def in_cps(j, b, s=s, m=m):
            p, gb = pg(j, s, m)
            rowa = jnp.minimum(gb * m, 1023)
            rowb = jnp.minimum((gb + 1) * m, 1023)
            return [
                pltpu.make_async_copy(
                    pe.at[pl.ds(p * _PLANE_SZ + rowa * 2048, 2048)],
                    rbufs[b].at[pl.ds(0, 2048)], sis[b]),
                pltpu.make_async_copy(
                    pe.at[pl.ds(p * _PLANE_SZ + rowb * 2048, 2048)],
                    rbufs[b].at[pl.ds(2048, 2048)], sis[b]),
            ]

        def out_cp(j, b, l=l, s=s, m=m):
            p, gb = pg(j, s, m)
            row0 = _LEVEL_BASE[l] + (p * s + gb) * s
            return pltpu.make_async_copy(
                obufs[b].at[pl.ds(0, s), :], qt.at[pl.ds(row0, s), :], sos[b])

        for cp in in_cps(0, 0):
            cp.start()
        for cp in in_cps(1, 1):
            cp.start()

        def pairloop(j2, carry, l=l, s=s, m=m, n_l=n_l):
            for b in range(2):
                j = j2 * 2 + b
                for cp in in_cps(j, b):
                    cp.wait()

                @pl.when(j >= 2)
                def _():
                    out_cp(j - 2, b).wait()

                def pair(i, carry2, b=b, m=m):
                    q = i * 2 + qdel
                    la = jnp.minimum((q + adel) * m, 1023)
                    idx = la * 2 + fplus + rowsel
                    plsc.store_scatter(obufs[b], [i * 2 + qdel, kcol],
                                       plsc.load_gather(rbufs[b], [idx]))
                    return carry2

                lax.fori_loop(0, s // 2, pair, 0, unroll=4)
                out_cp(j, b).start()

                @pl.when(j + 2 < n_l)
                def _():
                    for cp in in_cps(j + 2, b):
                        cp.start()
            return carry

        lax.fori_loop(0, n_l // 2, pairloop, 0)
        out_cp(n_l - 2, 0).wait()
        out_cp(n_l - 1, 1).wait()


DCH = 256
DNC = PPW // DCH


def _depad_body(posn, pflat, buf0, buf1, out0, out1, sem_i, sem_o):
    wid = lax.axis_index("s") * 2 + lax.axis_index("c")
    lane = lax.iota(jnp.int32, 16)
    base0 = wid * PPW
    bufs = (buf0, buf1)
    outs = (out0, out1)

    def in_copy(ch, slot):
        return pltpu.make_async_copy(
            posn.at[pl.ds(base0 + ch * DCH, DCH), :], bufs[slot], sem_i)

    def out_copy(ch, slot):
        return pltpu.make_async_copy(
            outs[slot],
            pflat.at[pl.ds((base0 + ch * DCH) * 3, DCH * 3)], sem_o)

    in_copy(0, 0).start()
    in_copy(1, 1).start()

    def pair(j, carry):
        for b in range(2):
            ch = j * 2 + b
            in_copy(ch, b).wait()

            @pl.when(ch >= 2)
            def _():
                out_copy(ch - 2, b).wait()

            for g in range(DCH // 16):
                pv = lane + g * 16
                for c in range(3):
                    v = plsc.load_gather(
                        bufs[b], [pv, jnp.full((16,), c, jnp.int32)])
                    plsc.store_scatter(
                        outs[b], [lane * 3 + (g * 48 + c)], v)
            out_copy(ch, b).start()

            @pl.when(ch + 2 < DNC)
            def _():
                in_copy(ch + 2, b).start()
        return carry

    lax.fori_loop(0, DNC // 2, pair, 0)
    out_copy(DNC - 2, 0).wait()
    out_copy(DNC - 1, 1).wait()


def _lookup_body(qtab, pos, out, pv0, pv1, ix0, ix1, gv0, gv1, ov0, ov1,
                 sp0, sp1, sg0, sg1, so0, so1):
    wid = lax.axis_index("s") * 2 + lax.axis_index("c")
    lane = lax.iota(jnp.int32, 16)
    lane3 = lane * 3
    lane32 = lane * 32
    base0 = wid * PPW
    pvs, ixs, gvs, ovs = (pv0, pv1), (ix0, ix1), (gv0, gv1), (ov0, ov1)
    sps, sgs, sos = (sp0, sp1), (sg0, sg1), (so0, so1)

    def pos_cp(ch, b):
        return pltpu.make_async_copy(
            pos.at[pl.ds((base0 + ch * C) * 3, C * 3)], pvs[b], sps[b])

    def out_cp(ch, b):
        return pltpu.make_async_copy(
            ovs[b], out.at[pl.ds((base0 + ch * C) * 32, C * 32)], sos[b])

    def gath_cps(b):
        return [pltpu.make_async_copy(
                    qtab.at[ixs[b].at[c]], gvs[b].at[pl.ds(c * C, C)], sgs[b])
                for c in range(12)]

    def phase_a(b):
        pv, ix = pvs[b], ixs[b]

        def ga_body(g, carry):
            x = plsc.load_gather(pv, [lane3 + g * 48])
            y = plsc.load_gather(pv, [lane3 + (g * 48 + 1)])
            z = plsc.load_gather(pv, [lane3 + (g * 48 + 2)])
            for l, s in enumerate(SCALES):
                pgs = []
                for v in (x, y, z):
                    p_ = v * (s - 1.0) + 0.5
                    pgs.append(p_.astype(jnp.int32))
                for p, (a, b2) in enumerate(((0, 1), (1, 2), (2, 0))):
                    row = pgs[b2] * s + pgs[a] + (_LEVEL_BASE[l] + p * s * s)
                    ix[l * 3 + p, pl.ds(g * 16, 16)] = row
            return carry

        lax.fori_loop(0, NG, ga_body, 0)

    def phase_c(b):
        pv, gv, ov = pvs[b], gvs[b], ovs[b]

        def gc_body(g, carry):
            x = plsc.load_gather(pv, [lane3 + g * 48])
            y = plsc.load_gather(pv, [lane3 + (g * 48 + 1)])
            z = plsc.load_gather(pv, [lane3 + (g * 48 + 2)])
            pt = lane + g * 16
            for l, s in enumerate(SCALES):
                fr = []
                for v in (x, y, z):
                    p_ = v * (s - 1.0) + 0.5
                    fr.append(p_ - p_.astype(jnp.int32).astype(jnp.float32))
                feats = []
                for p, (a, b2) in enumerate(((0, 1), (1, 2), (2, 0))):
                    c = l * 3 + p
                    rvec = pt + c * C
                    k = [plsc.load_gather(
                            gv, [rvec, jnp.full((16,), kk, jnp.int32)])
                         for kk in range(8)]
                    fa, fb = fr[a], fr[b2]
                    m00 = k[0] + fa * (k[2] - k[0])
                    m01 = k[1] + fa * (k[3] - k[1])
                    m10 = k[4] + fa * (k[6] - k[4])
                    m11 = k[5] + fa * (k[7] - k[5])
                    feats.append((m00 + fb * (m10 - m00),
                                  m01 + fb * (m11 - m01)))
                prod0 = feats[0][0] * feats[1][0] * feats[2][0]
                prod1 = feats[0][1] * feats[1][1] * feats[2][1]
                cols = [feats[0][0], feats[0][1], feats[1][0], feats[1][1],
                        feats[2][0], feats[2][1], prod0, prod1]
                for ci, vec in enumerate(cols):
                    plsc.store_scatter(
                        ov, [lane32 + (g * 512 + l * 8 + ci)], vec)
            return carry

        lax.fori_loop(0, NG, gc_body, 0)

    pos_cp(0, 0).start()
    pos_cp(0, 0).wait()
    phase_a(0)
    for cp in gath_cps(0):
        cp.start()
    pos_cp(1, 1).start()

    def pair(j, carry):
        for b2 in range(2):
            ch = 1 + j * 2 + b2
            s = (1 + b2) % 2
            o = 1 - s
            pos_cp(ch, s).wait()
            phase_a(s)
            for cp in gath_cps(s):
                cp.start()
            for cp in gath_cps(o):
                cp.wait()

            @pl.when(ch >= 3)
            def _():
                out_cp(ch - 3, o).wait()

            phase_c(o)
            out_cp(ch - 1, o).start()
            pos_cp(ch + 1, o).start()
        return carry

    lax.fori_loop(0, (NCHUNK - 2) // 2, pair, 0)
    last = NCHUNK - 1
    pos_cp(last, 1).wait()
    phase_a(1)
    for cp in gath_cps(1):
        cp.start()
    for cp in gath_cps(0):
        cp.wait()
    out_cp(last - 3, 0).wait()
    phase_c(0)
    out_cp(last - 1, 0).start()
    for cp in gath_cps(1):
        cp.wait()
    out_cp(last - 2, 1).wait()
    phase_c(1)
    out_cp(last, 1).start()
    out_cp(last - 1, 0).wait()
    out_cp(last, 1).wait()


@functools.partial(jax.jit)
def kernel(positions, plane_embedding):
    mesh = plsc.VectorSubcoreMesh(core_axis_name="c", subcore_axis_name="s")
    cparams = pltpu.CompilerParams(
        needs_layout_passes=False, use_tc_tiling_on_sc=False)
    build = pl.kernel(
        _build_body,
        out_type=jax.ShapeDtypeStruct((R_TOTAL, 8), jnp.float32),
        mesh=mesh,
        compiler_params=cparams,
        scratch_types=[
            pltpu.VMEM((4096,), jnp.float32),
            pltpu.VMEM((4096,), jnp.float32),
            pltpu.VMEM((1024, 8), jnp.float32),
            pltpu.VMEM((1024, 8), jnp.float32),
            pltpu.SemaphoreType.DMA,
            pltpu.SemaphoreType.DMA,
            pltpu.SemaphoreType.DMA,
            pltpu.SemaphoreType.DMA,
        ],
    )
    lookup = pl.kernel(
        _lookup_body,
        out_type=jax.ShapeDtypeStruct((N_PTS * 32,), jnp.float32),
        mesh=mesh,
        compiler_params=cparams,
        scratch_types=[
            pltpu.VMEM((3 * C,), jnp.float32),
            pltpu.VMEM((3 * C,), jnp.float32),
            pltpu.VMEM((12, C), jnp.int32),
            pltpu.VMEM((12, C), jnp.int32),
            pltpu.VMEM((12 * C, 8), jnp.float32),
            pltpu.VMEM((12 * C, 8), jnp.float32),
            pltpu.VMEM((C * 32,), jnp.float32),
            pltpu.VMEM((C * 32,), jnp.float32),
            pltpu.SemaphoreType.DMA,
            pltpu.SemaphoreType.DMA,
            pltpu.SemaphoreType.DMA,
            pltpu.SemaphoreType.DMA,
            pltpu.SemaphoreType.DMA,
            pltpu.SemaphoreType.DMA,
        ],
    )
    depad = pl.kernel(
        _depad_body,
        out_type=jax.ShapeDtypeStruct((N_PTS * 3,), jnp.float32),
        mesh=mesh,
        compiler_params=pltpu.CompilerParams(
            needs_layout_passes=False, use_tc_tiling_on_sc=True),
        scratch_types=[
            pltpu.VMEM((DCH, 3), jnp.float32),
            pltpu.VMEM((DCH, 3), jnp.float32),
            pltpu.VMEM((DCH * 3,), jnp.float32),
            pltpu.VMEM((DCH * 3,), jnp.float32),
            pltpu.SemaphoreType.DMA,
            pltpu.SemaphoreType.DMA,
        ],
    )
    qtab = build(plane_embedding)
    return lookup(qtab, depad(positions)).reshape(N_PTS, 32)


# speedup vs baseline: 61.7953x; 1.0002x over previous
import functools

import jax
import jax.numpy as jnp
from jax import lax
from jax.experimental import pallas as pl
from jax.experimental.pallas import tpu as pltpu
from jax.experimental.pallas import tpu_sc as plsc

PLANE_RES = 1024
FEAT = 2
N_PTS = 1048576
SCALES = (128, 256, 512, 1024)
NW = 32
PPW = N_PTS // NW
C = 128
NG = C // 16
NCHUNK = PPW // C

_LEVEL_BASE = []
_acc = 0
for _s in SCALES:
    _LEVEL_BASE.append(_acc)
    _acc += 3 * _s * _s
R_TOTAL = _acc

_PLANE_SZ = PLANE_RES * PLANE_RES * FEAT


def _build_body(pe, qt, ra0, ra1, ob0, ob1, sem_i0, sem_i1, sem_o0, sem_o1):
    wid = lax.axis_index("s") * 2 + lax.axis_index("c")
    lane = lax.iota(jnp.int32, 16)
    qdel = lane // 8
    adel = (lane // 2) % 2
    fplus = (lane % 2) + (lane // 8) * 0
    rowsel = (lane % 8) // 4 * 2048

    kcol = lane % 8
    rbufs = (ra0, ra1)
    obufs = (ob0, ob1)
    sis = (sem_i0, sem_i1)
    sos = (sem_o0, sem_o1)

    for l, s in enumerate(SCALES):
        m = PLANE_RES // s
        n_l = 3 * s // NW

        def pg(j, s=s, m=m):
            sid = wid * n_l + j
            p = sid // s
            gb = sid % s
            return p, gb

        def in_cps(j, b, s=s, m=m):
            p, gb = pg(j, s, m)
            rowa = jnp.minimum(gb * m, 1023)
            rowb = jnp.minimum((gb + 1) * m, 1023)
            return [
                pltpu.make_async_copy(
                    pe.at[pl.ds(p * _PLANE_SZ + rowa * 2048, 2048)],
                    rbufs[b].at[pl.ds(0, 2048)], sis[b]),
                pltpu.make_async_copy(
                    pe.at[pl.ds(p * _PLANE_SZ + rowb * 2048, 2048)],
                    rbufs[b].at[pl.ds(2048, 2048)], sis[b]),
            ]

        def out_cp(j, b, l=l, s=s, m=m):
            p, gb = pg(j, s, m)
            row0 = _LEVEL_BASE[l] + (p * s + gb) * s
            return pltpu.make_async_copy(
                obufs[b].at[pl.ds(0, s), :], qt.at[pl.ds(row0, s), :], sos[b])

        for cp in in_cps(0, 0):
            cp.start()
        for cp in in_cps(1, 1):
            cp.start()

        def pairloop(j2, carry, l=l, s=s, m=m, n_l=n_l):
            for b in range(2):
                j = j2 * 2 + b
                for cp in in_cps(j, b):
                    cp.wait()

                @pl.when(j >= 2)
                def _():
                    out_cp(j - 2, b).wait()

                def pair(i, carry2, b=b, m=m):
                    q = i * 2 + qdel
                    la = jnp.minimum((q + adel) * m, 1023)
                    idx = la * 2 + fplus + rowsel
                    plsc.store_scatter(obufs[b], [i * 2 + qdel, kcol],
                                       plsc.load_gather(rbufs[b], [idx]))
                    return carry2

                lax.fori_loop(0, s // 2, pair, 0, unroll=4)
                out_cp(j, b).start()

                @pl.when(j + 2 < n_l)
                def _():
                    for cp in in_cps(j + 2, b):
                        cp.start()
            return carry

        lax.fori_loop(0, n_l // 2, pairloop, 0)
        out_cp(n_l - 2, 0).wait()
        out_cp(n_l - 1, 1).wait()


DCH = 256
DNC = PPW // DCH


def _depad_body(posn, pflat, buf0, buf1, out0, out1, sem_i, sem_o):
    wid = lax.axis_index("s") * 2 + lax.axis_index("c")
    lane = lax.iota(jnp.int32, 16)
    base0 = wid * PPW
    bufs = (buf0, buf1)
    outs = (out0, out1)

    def in_copy(ch, slot):
        return pltpu.make_async_copy(
            posn.at[pl.ds(base0 + ch * DCH, DCH), :], bufs[slot], sem_i)

    def out_copy(ch, slot):
        return pltpu.make_async_copy(
            outs[slot],
            pflat.at[pl.ds((base0 + ch * DCH) * 3, DCH * 3)], sem_o)

    in_copy(0, 0).start()
    in_copy(1, 1).start()

    def pair(j, carry):
        for b in range(2):
            ch = j * 2 + b
            in_copy(ch, b).wait()

            @pl.when(ch >= 2)
            def _():
                out_copy(ch - 2, b).wait()

            for g in range(DCH // 16):
                pv = lane + g * 16
                for c in range(3):
                    v = plsc.load_gather(
                        bufs[b], [pv, jnp.full((16,), c, jnp.int32)])
                    plsc.store_scatter(
                        outs[b], [lane * 3 + (g * 48 + c)], v)
            out_copy(ch, b).start()

            @pl.when(ch + 2 < DNC)
            def _():
                in_copy(ch + 2, b).start()
        return carry

    lax.fori_loop(0, DNC // 2, pair, 0)
    out_copy(DNC - 2, 0).wait()
    out_copy(DNC - 1, 1).wait()


def _lookup_body(qtab, pos, out, pv0, pv1, ix0, ix1, gv0, gv1, ov0, ov1,
                 sp0, sp1, sg0, sg1, so0, so1):
    wid = lax.axis_index("s") * 2 + lax.axis_index("c")
    lane = lax.iota(jnp.int32, 16)
    lane3 = lane * 3
    lane32 = lane * 32
    base0 = wid * PPW
    pvs, ixs, gvs, ovs = (pv0, pv1), (ix0, ix1), (gv0, gv1), (ov0, ov1)
    sps, sgs, sos = (sp0, sp1), (sg0, sg1), (so0, so1)

    def pos_cp(ch, b):
        return pltpu.make_async_copy(
            pos.at[pl.ds((base0 + ch * C) * 3, C * 3)], pvs[b], sps[b])

    def out_cp(ch, b):
        return pltpu.make_async_copy(
            ovs[b], out.at[pl.ds((base0 + ch * C) * 32, C * 32)], sos[b])

    def gath_cps(b):
        return [pltpu.make_async_copy(
                    qtab.at[ixs[b].at[c]], gvs[b].at[pl.ds(c * C, C)], sgs[b])
                for c in range(12)]

    def phase_a(b):
        pv, ix = pvs[b], ixs[b]

        def ga_body(g, carry):
            x = plsc.load_gather(pv, [lane3 + g * 48])
            y = plsc.load_gather(pv, [lane3 + (g * 48 + 1)])
            z = plsc.load_gather(pv, [lane3 + (g * 48 + 2)])
            for l, s in enumerate(SCALES):
                pgs = []
                for v in (x, y, z):
                    p_ = v * (s - 1.0) + 0.5
                    pgs.append(p_.astype(jnp.int32))
                for p, (a, b2) in enumerate(((0, 1), (1, 2), (2, 0))):
                    row = pgs[b2] * s + pgs[a] + (_LEVEL_BASE[l] + p * s * s)
                    ix[l * 3 + p, pl.ds(g * 16, 16)] = row
            return carry

        lax.fori_loop(0, NG, ga_body, 0, unroll=2)

    lane8 = lane * 8

    def phase_c(b):
        pv, ov = pvs[b], ovs[b]
        gv = gvs[b]

        def gc_body(g, carry):
            x = plsc.load_gather(pv, [lane3 + g * 48])
            y = plsc.load_gather(pv, [lane3 + (g * 48 + 1)])
            z = plsc.load_gather(pv, [lane3 + (g * 48 + 2)])
            pt = lane + g * 16
            for l, s in enumerate(SCALES):
                fr = []
                for v in (x, y, z):
                    p_ = v * (s - 1.0) + 0.5
                    fr.append(p_ - p_.astype(jnp.int32).astype(jnp.float32))
                feats = []
                for p, (a, b2) in enumerate(((0, 1), (1, 2), (2, 0))):
                    c = l * 3 + p
                    rvec = pt + c * C
                    k = [plsc.load_gather(
                            gv, [rvec, jnp.full((16,), kk, jnp.int32)])
                         for kk in range(8)]
                    fa, fb = fr[a], fr[b2]
                    m00 = k[0] + fa * (k[2] - k[0])
                    m01 = k[1] + fa * (k[3] - k[1])
                    m10 = k[4] + fa * (k[6] - k[4])
                    m11 = k[5] + fa * (k[7] - k[5])
                    feats.append((m00 + fb * (m10 - m00),
                                  m01 + fb * (m11 - m01)))
                prod0 = feats[0][0] * feats[1][0] * feats[2][0]
                prod1 = feats[0][1] * feats[1][1] * feats[2][1]
                cols = [feats[0][0], feats[0][1], feats[1][0], feats[1][1],
                        feats[2][0], feats[2][1], prod0, prod1]
                for ci, vec in enumerate(cols):
                    plsc.store_scatter(
                        ov, [lane32 + (g * 512 + l * 8 + ci)], vec)
            return carry

        lax.fori_loop(0, NG, gc_body, 0, unroll=2)

    pos_cp(0, 0).start()
    pos_cp(0, 0).wait()
    phase_a(0)
    for cp in gath_cps(0):
        cp.start()
    pos_cp(1, 1).start()

    def pair(j, carry):
        for b2 in range(2):
            ch = 1 + j * 2 + b2
            s = (1 + b2) % 2
            o = 1 - s
            pos_cp(ch, s).wait()
            phase_a(s)
            for cp in gath_cps(s):
                cp.start()
            for cp in gath_cps(o):
                cp.wait()

            @pl.when(ch >= 3)
            def _():
                out_cp(ch - 3, o).wait()

            phase_c(o)
            out_cp(ch - 1, o).start()
            pos_cp(ch + 1, o).start()
        return carry

    lax.fori_loop(0, (NCHUNK - 2) // 2, pair, 0)
    last = NCHUNK - 1
    pos_cp(last, 1).wait()
    phase_a(1)
    for cp in gath_cps(1):
        cp.start()
    for cp in gath_cps(0):
        cp.wait()
    out_cp(last - 3, 0).wait()
    phase_c(0)
    out_cp(last - 1, 0).start()
    for cp in gath_cps(1):
        cp.wait()
    out_cp(last - 2, 1).wait()
    phase_c(1)
    out_cp(last, 1).start()
    out_cp(last - 1, 0).wait()
    out_cp(last, 1).wait()


@functools.partial(jax.jit)
def kernel(positions, plane_embedding):
    mesh = plsc.VectorSubcoreMesh(core_axis_name="c", subcore_axis_name="s")
    cparams = pltpu.CompilerParams(
        needs_layout_passes=False, use_tc_tiling_on_sc=False)
    build = pl.kernel(
        _build_body,
        out_type=jax.ShapeDtypeStruct((R_TOTAL, 8), jnp.float32),
        mesh=mesh,
        compiler_params=cparams,
        scratch_types=[
            pltpu.VMEM((4096,), jnp.float32),
            pltpu.VMEM((4096,), jnp.float32),
            pltpu.VMEM((1024, 8), jnp.float32),
            pltpu.VMEM((1024, 8), jnp.float32),
            pltpu.SemaphoreType.DMA,
            pltpu.SemaphoreType.DMA,
            pltpu.SemaphoreType.DMA,
            pltpu.SemaphoreType.DMA,
        ],
    )
    lookup = pl.kernel(
        _lookup_body,
        out_type=jax.ShapeDtypeStruct((N_PTS * 32,), jnp.float32),
        mesh=mesh,
        compiler_params=cparams,
        scratch_types=[
            pltpu.VMEM((3 * C,), jnp.float32),
            pltpu.VMEM((3 * C,), jnp.float32),
            pltpu.VMEM((12, C), jnp.int32),
            pltpu.VMEM((12, C), jnp.int32),
            pltpu.VMEM((12 * C, 8), jnp.float32),
            pltpu.VMEM((12 * C, 8), jnp.float32),
            pltpu.VMEM((C * 32,), jnp.float32),
            pltpu.VMEM((C * 32,), jnp.float32),
            pltpu.SemaphoreType.DMA,
            pltpu.SemaphoreType.DMA,
            pltpu.SemaphoreType.DMA,
            pltpu.SemaphoreType.DMA,
            pltpu.SemaphoreType.DMA,
            pltpu.SemaphoreType.DMA,
        ],
    )
    depad = pl.kernel(
        _depad_body,
        out_type=jax.ShapeDtypeStruct((N_PTS * 3,), jnp.float32),
        mesh=mesh,
        compiler_params=pltpu.CompilerParams(
            needs_layout_passes=False, use_tc_tiling_on_sc=True),
        scratch_types=[
            pltpu.VMEM((DCH, 3), jnp.float32),
            pltpu.VMEM((DCH, 3), jnp.float32),
            pltpu.VMEM((DCH * 3,), jnp.float32),
            pltpu.VMEM((DCH * 3,), jnp.float32),
            pltpu.SemaphoreType.DMA,
            pltpu.SemaphoreType.DMA,
        ],
    )
    qtab = build(plane_embedding)
    return lookup(qtab, depad(positions)).reshape(N_PTS, 32)
